# baseline (speedup 1.0000x reference)
import functools

import jax
import jax.numpy as jnp
from jax import lax
from jax.experimental import pallas as pl
from jax.experimental.pallas import tpu as pltpu
from jax.experimental.pallas import tpu_sc as plsc

NC = 2
NS = 16
NW = NC * NS
CH = 128
MAX_NORM = 1.0
EPS = 1e-7


def _renorm_rows(buf, n_rows, d):
    n_slices = d // 16

    def row_body(r, _):
        parts = [buf[r, pl.ds(16 * c, 16)] for c in range(n_slices)]
        ss_vec = parts[0] * parts[0]
        for c in range(1, n_slices):
            ss_vec = ss_vec + parts[c] * parts[c]
        lanes = lax.iota(jnp.int32, 16)
        ssb = ss_vec
        for sh in (8, 4, 2, 1):
            perm = lanes ^ sh
            ssb = ssb + ssb.at[perm].get(mode="promise_in_bounds")
        i = lax.bitcast_convert_type(ssb, jnp.int32)
        i = jnp.int32(0x5F3759DF) - (i >> 1)
        y = lax.bitcast_convert_type(i, jnp.float32)
        half_ss = 0.5 * ssb
        for _ in range(3):
            y = y * (1.5 - half_ss * y * y)
        norm = ssb * y
        recip = MAX_NORM / (norm + EPS)
        scale = jnp.where(ssb > MAX_NORM * MAX_NORM, recip, 1.0)
        for c in range(n_slices):
            buf[r, pl.ds(16 * c, 16)] = parts[c] * scale
        return 0

    lax.fori_loop(0, n_rows, row_body, 0)


def kernel(input, speakers, lut_p, lut_s):
    B, L = input.shape
    V, D = lut_p.shape
    N = B * L
    per_w = N // NW
    n_chunks = per_w // CH
    spk_per_w = B // NW
    assert per_w * NW == N and n_chunks * CH == per_w and spk_per_w * NW == B
    assert spk_per_w <= CH

    idx = input.reshape(NW, n_chunks, CH).astype(jnp.int32)
    spk = speakers.reshape(NW, spk_per_w).astype(jnp.int32)

    mesh = plsc.VectorSubcoreMesh(core_axis_name="c", subcore_axis_name="s")

    @functools.partial(
        pl.kernel,
        mesh=mesh,
        compiler_params=pltpu.CompilerParams(use_tc_tiling_on_sc=False),
        out_type=[
            jax.ShapeDtypeStruct((N, D), jnp.float32),
            jax.ShapeDtypeStruct((B, D), jnp.float32),
        ],
        scratch_types=[
            pltpu.VMEM((n_chunks, CH), jnp.int32),
            pltpu.VMEM((spk_per_w,), jnp.int32),
            pltpu.VMEM((CH, D), jnp.float32),
            pltpu.SemaphoreType.DMA,
        ],
    )
    def run(idx_hbm, spk_hbm, lut_p_hbm, lut_s_hbm, out_hbm, ident_hbm,
            idx_v, spk_v, rows_v, gsem):
        cid = lax.axis_index("c")
        sid = lax.axis_index("s")
        wid = sid * NC + cid
        base = wid * per_w

        pltpu.sync_copy(idx_hbm.at[wid], idx_v)
        pltpu.sync_copy(spk_hbm.at[wid], spk_v)

        def chunk_body(g, _):
            pltpu.async_copy(lut_p_hbm.at[idx_v.at[g]], rows_v, gsem).wait()
            _renorm_rows(rows_v, CH, D)
            pltpu.sync_copy(rows_v, out_hbm.at[pl.ds(base + g * CH, CH)])
            return 0

        lax.fori_loop(0, n_chunks, chunk_body, 0)

        spk_rows = rows_v.at[pl.ds(0, spk_per_w)]
        pltpu.async_copy(lut_s_hbm.at[spk_v], spk_rows, gsem).wait()
        _renorm_rows(rows_v, spk_per_w, D)
        pltpu.sync_copy(spk_rows, ident_hbm.at[pl.ds(wid * spk_per_w, spk_per_w)])

    out, ident = run(idx, spk, lut_p, lut_s)
    return out.reshape(B, L, D), ident


# speedup vs baseline: 1.8238x; 1.8238x over previous
import functools

import jax
import jax.numpy as jnp
from jax import lax
from jax.experimental import pallas as pl
from jax.experimental.pallas import tpu as pltpu
from jax.experimental.pallas import tpu_sc as plsc

NC = 2
NS = 16
NW = NC * NS
CH = 128
NBUF = 4
GRP = 8
MAX_NORM = 1.0
EPS = 1e-7


def _renorm_chunk(buf, n_rows, d):
    n_slices = d // 16
    lanes = lax.iota(jnp.int32, 16)

    def group_body(gi, _):
        r0 = gi * GRP
        parts = []
        accs = []
        for j in range(GRP):
            p = [buf[r0 + j, pl.ds(16 * c, 16)] for c in range(n_slices)]
            parts.append(p)
            sv = p[0] * p[0]
            for c in range(1, n_slices):
                sv = sv + p[c] * p[c]
            for sh in (8, 4, 2, 1):
                sv = sv + sv.at[lanes ^ sh].get(mode="promise_in_bounds")
            accs.append(jnp.where(lanes == j, sv, 0.0))
        while len(accs) > 1:
            accs = [accs[k] + accs[k + 1] for k in range(0, len(accs), 2)]
        acc = accs[0]
        iv = lax.bitcast_convert_type(acc, jnp.int32)
        iv = jnp.int32(0x5F3759DF) - (iv >> 1)
        y = lax.bitcast_convert_type(iv, jnp.float32)
        half = 0.5 * acc
        for _ in range(3):
            y = y * (1.5 - half * y * y)
        scale = jnp.where(acc > MAX_NORM * MAX_NORM, y * (1.0 - EPS * y), 1.0)
        for j in range(GRP):
            sj = scale.at[jnp.full((16,), j, jnp.int32)].get(
                mode="promise_in_bounds")
            for c in range(n_slices):
                buf[r0 + j, pl.ds(16 * c, 16)] = parts[j][c] * sj
        return 0

    lax.fori_loop(0, n_rows // GRP, group_body, 0)


def kernel(input, speakers, lut_p, lut_s):
    B, L = input.shape
    V, D = lut_p.shape
    N = B * L
    per_w = N // NW
    n_chunks = per_w // CH
    spk_per_w = B // NW
    assert per_w * NW == N and n_chunks * CH == per_w and spk_per_w * NW == B
    assert spk_per_w <= CH and n_chunks % NBUF == 0 and n_chunks >= 3 * NBUF

    idx = input.reshape(NW, n_chunks, CH).astype(jnp.int32)
    spk = speakers.reshape(NW, spk_per_w).astype(jnp.int32)

    mesh = plsc.VectorSubcoreMesh(core_axis_name="c", subcore_axis_name="s")

    @functools.partial(
        pl.kernel,
        mesh=mesh,
        compiler_params=pltpu.CompilerParams(use_tc_tiling_on_sc=False),
        out_type=[
            jax.ShapeDtypeStruct((N, D), jnp.float32),
            jax.ShapeDtypeStruct((B, D), jnp.float32),
        ],
        scratch_types=[
            pltpu.VMEM((n_chunks, CH), jnp.int32),
            pltpu.VMEM((spk_per_w,), jnp.int32),
            pltpu.VMEM((NBUF, CH, D), jnp.float32),
            pltpu.VMEM((spk_per_w, D), jnp.float32),
            pltpu.SemaphoreType.DMA((NBUF,)),
            pltpu.SemaphoreType.DMA((NBUF,)),
            pltpu.SemaphoreType.DMA,
        ],
    )
    def run(idx_hbm, spk_hbm, lut_p_hbm, lut_s_hbm, out_hbm, ident_hbm,
            idx_v, spk_v, rows_v, srows_v, gsem, osem, ssem):
        cid = lax.axis_index("c")
        sid = lax.axis_index("s")
        wid = sid * NC + cid
        base = wid * per_w

        pltpu.sync_copy(idx_hbm.at[wid], idx_v)
        pltpu.sync_copy(spk_hbm.at[wid], spk_v)

        pltpu.async_copy(lut_s_hbm.at[spk_v], srows_v, ssem).wait()
        _renorm_chunk(srows_v, spk_per_w, D)
        pltpu.sync_copy(
            srows_v, ident_hbm.at[pl.ds(wid * spk_per_w, spk_per_w)])

        def start_gather(g, p):
            pltpu.async_copy(
                lut_p_hbm.at[idx_v.at[g]], rows_v.at[p], gsem.at[p])

        def wait_gather(g, p):
            pltpu.make_async_copy(
                lut_p_hbm.at[idx_v.at[g]], rows_v.at[p], gsem.at[p]).wait()

        def start_write(g, p):
            pltpu.async_copy(
                rows_v.at[p], out_hbm.at[pl.ds(base + g * CH, CH)],
                osem.at[p])

        def wait_write(g, p):
            pltpu.make_async_copy(
                rows_v.at[p], out_hbm.at[pl.ds(base + g * CH, CH)],
                osem.at[p]).wait()

        def body(g, p, first, last):
            wait_gather(g, p)
            _renorm_chunk(rows_v.at[p], CH, D)
            start_write(g, p)
            h = g + 2
            q = (p + 2) % NBUF
            if not last:
                if not first:
                    wait_write(h - NBUF, q)
                start_gather(h, q)

        start_gather(0, 0)
        start_gather(1, 1)

        for p in range(NBUF):
            body(p, p, first=(p < 2), last=False)

        def round_body(i, _):
            g0 = i * NBUF
            for p in range(NBUF):
                body(g0 + p, p, first=False, last=False)
            return 0

        lax.fori_loop(1, n_chunks // NBUF - 1, round_body, 0)

        g0 = n_chunks - NBUF
        for p in range(NBUF):
            body(g0 + p, p, first=False, last=(p >= 2))

        for p in range(NBUF):
            wait_write(g0 + p, p)

    out, ident = run(idx, spk, lut_p, lut_s)
    return out.reshape(B, L, D), ident
